# baseline (speedup 1.0000x reference)
import functools

import jax
import jax.numpy as jnp
from jax import lax
from jax.experimental import pallas as pl
from jax.experimental.pallas import tpu as pltpu
from jax.experimental.pallas import tpu_sc as plsc

N = 10000
E = 160000
NP = 10240
EP = 163840
CHUNK = 128
CPT = (EP // 32) // CHUNK
ROWS_PT = NP // 16

_mesh = plsc.VectorSubcoreMesh(core_axis_name="c", subcore_axis_name="s")
_f32 = jnp.float32
_sc_params = pltpu.CompilerParams(use_tc_tiling_on_sc=False)


@functools.partial(
    pl.kernel,
    out_type=jax.ShapeDtypeStruct((2, NP, 16), _f32),
    mesh=_mesh,
    scratch_types=[
        pltpu.VMEM((CPT, CHUNK), jnp.int32),
        pltpu.VMEM((CHUNK, 16), _f32),
        pltpu.VMEM_SHARED((NP, 16), _f32),
        pltpu.SemaphoreType.DMA,
    ],
    compiler_params=_sc_params,
)
def _deg_sc(dst_hbm, ones_hbm, zeros_hbm, out_hbm, idx_v, ones_v, acc, sem):
    cid = lax.axis_index("c")
    sid = lax.axis_index("s")
    tid = cid * 16 + sid
    pltpu.sync_copy(dst_hbm.at[pl.ds(tid * CPT, CPT)], idx_v)
    pltpu.sync_copy(ones_hbm, ones_v)
    pltpu.sync_copy(zeros_hbm, acc.at[pl.ds(sid * ROWS_PT, ROWS_PT)])
    plsc.subcore_barrier()

    @pl.loop(0, CPT, step=8)
    def _(j):
        for k in range(8):
            pltpu.async_copy(ones_v, acc.at[idx_v.at[j + k]], sem, add=True)
        for k in range(8):
            pltpu.make_async_copy(ones_v, acc.at[idx_v.at[j + k]], sem).wait()

    plsc.subcore_barrier()
    pltpu.sync_copy(
        acc.at[pl.ds(sid * ROWS_PT, ROWS_PT)],
        out_hbm.at[cid, pl.ds(sid * ROWS_PT, ROWS_PT)],
    )


def _make_agg(d):
    @functools.partial(
        pl.kernel,
        out_type=jax.ShapeDtypeStruct((2, NP, d), _f32),
        mesh=_mesh,
        scratch_types=[
            pltpu.VMEM((CPT, CHUNK), jnp.int32),
            pltpu.VMEM((CPT, CHUNK), jnp.int32),
            [pltpu.VMEM((CHUNK, d), _f32)] * 8,
            [pltpu.SemaphoreType.DMA] * 8,
            [pltpu.SemaphoreType.DMA] * 8,
            pltpu.VMEM_SHARED((NP, d), _f32),
        ],
        compiler_params=_sc_params,
    )
    def _agg(y_hbm, src_hbm, dst_hbm, zeros_hbm, out_hbm,
             srcv, dstv, bufs, gsems, ssems, acc):
        cid = lax.axis_index("c")
        sid = lax.axis_index("s")
        tid = cid * 16 + sid
        pltpu.sync_copy(src_hbm.at[pl.ds(tid * CPT, CPT)], srcv)
        pltpu.sync_copy(dst_hbm.at[pl.ds(tid * CPT, CPT)], dstv)
        pltpu.sync_copy(zeros_hbm, acc.at[pl.ds(sid * ROWS_PT, ROWS_PT)])
        plsc.subcore_barrier()

        for c in range(4):
            pltpu.async_copy(y_hbm.at[srcv.at[c]], bufs[c], gsems[c])

        @pl.loop(0, CPT, step=8)
        def _(j):
            for k in range(8):
                b = k % 8
                pltpu.make_async_copy(y_hbm.at[srcv.at[j + k]],
                                      bufs[b], gsems[b]).wait()
                pltpu.async_copy(bufs[b], acc.at[dstv.at[j + k]],
                                 ssems[b], add=True)
                bn = (k + 4) % 8

                @pl.when(j + k + 4 < CPT)
                def _():
                    @pl.when(j + k >= 4)
                    def _():
                        pltpu.make_async_copy(
                            bufs[bn], acc.at[dstv.at[j + k - 4]],
                            ssems[bn]).wait()

                    pltpu.async_copy(y_hbm.at[srcv.at[j + k + 4]],
                                     bufs[bn], gsems[bn])

        for c in range(CPT - 8, CPT):
            b = c % 8
            pltpu.make_async_copy(bufs[b], acc.at[dstv.at[c]],
                                  ssems[b]).wait()

        plsc.subcore_barrier()
        pltpu.sync_copy(
            acc.at[pl.ds(sid * ROWS_PT, ROWS_PT)],
            out_hbm.at[cid, pl.ds(sid * ROWS_PT, ROWS_PT)],
        )

    return _agg


_agg64 = _make_agg(64)
_agg32 = _make_agg(32)


_BLK = 2048
_GRID = NP // _BLK


def _dis_invdeg(dp_ref):
    deg = 1.0 + dp_ref[0, :, 0:1] + dp_ref[1, :, 0:1]
    dis = 1.0 / jnp.sqrt(deg)
    return dis, dis * dis


def _dense1_body(x_ref, w_ref, dp_ref, y_ref, s_ref):
    xw = jnp.dot(x_ref[...], w_ref[...], preferred_element_type=_f32,
                 precision=lax.Precision.HIGHEST)
    dis, invd = _dis_invdeg(dp_ref)
    y_ref[...] = xw * dis
    s_ref[...] = xw * invd


def _dense1(x, w1, dp):
    return pl.pallas_call(
        _dense1_body,
        grid=(_GRID,),
        in_specs=[
            pl.BlockSpec((_BLK, 256), lambda i: (i, 0)),
            pl.BlockSpec((256, 64), lambda i: (0, 0)),
            pl.BlockSpec((2, _BLK, 16), lambda i: (0, i, 0)),
        ],
        out_specs=[
            pl.BlockSpec((_BLK, 64), lambda i: (i, 0)),
            pl.BlockSpec((_BLK, 64), lambda i: (i, 0)),
        ],
        out_shape=[
            jax.ShapeDtypeStruct((N, 64), _f32),
            jax.ShapeDtypeStruct((N, 64), _f32),
        ],
    )(x, w1, dp)


def _dense2_body(ap_ref, s1_ref, dp_ref, w_ref, b_ref, y_ref, s_ref):
    dis, invd = _dis_invdeg(dp_ref)
    agg = ap_ref[0] + ap_ref[1]
    h = jnp.maximum(agg * dis + s1_ref[...] + b_ref[...], 0.0)
    xw = jnp.dot(h, w_ref[...], preferred_element_type=_f32,
                 precision=lax.Precision.HIGHEST)
    y_ref[...] = xw * dis
    s_ref[...] = xw * invd


def _dense2(ap, s1, dp, w2, b1):
    return pl.pallas_call(
        _dense2_body,
        grid=(_GRID,),
        in_specs=[
            pl.BlockSpec((2, _BLK, 64), lambda i: (0, i, 0)),
            pl.BlockSpec((_BLK, 64), lambda i: (i, 0)),
            pl.BlockSpec((2, _BLK, 16), lambda i: (0, i, 0)),
            pl.BlockSpec((64, 32), lambda i: (0, 0)),
            pl.BlockSpec((1, 64), lambda i: (0, 0)),
        ],
        out_specs=[
            pl.BlockSpec((_BLK, 32), lambda i: (i, 0)),
            pl.BlockSpec((_BLK, 32), lambda i: (i, 0)),
        ],
        out_shape=[
            jax.ShapeDtypeStruct((N, 32), _f32),
            jax.ShapeDtypeStruct((N, 32), _f32),
        ],
    )(ap, s1, dp, w2, b1)


def _dense3_body(ap_ref, s2_ref, dp_ref, wfc_ref, b2_ref, bfc_ref, o_ref):
    dis, _ = _dis_invdeg(dp_ref)
    agg = ap_ref[0] + ap_ref[1]
    h = jnp.maximum(agg * dis + s2_ref[...] + b2_ref[...], 0.0)
    o_ref[...] = jnp.sum(h * wfc_ref[...], axis=1, keepdims=True) + bfc_ref[...]


def _dense3(ap, s2, dp, wfc_row, b2, bfc):
    return pl.pallas_call(
        _dense3_body,
        grid=(_GRID,),
        in_specs=[
            pl.BlockSpec((2, _BLK, 32), lambda i: (0, i, 0)),
            pl.BlockSpec((_BLK, 32), lambda i: (i, 0)),
            pl.BlockSpec((2, _BLK, 16), lambda i: (0, i, 0)),
            pl.BlockSpec((1, 32), lambda i: (0, 0)),
            pl.BlockSpec((1, 32), lambda i: (0, 0)),
            pl.BlockSpec((1, 1), lambda i: (0, 0)),
        ],
        out_specs=pl.BlockSpec((_BLK, 1), lambda i: (i, 0)),
        out_shape=jax.ShapeDtypeStruct((N, 1), _f32),
    )(ap, s2, dp, wfc_row, b2, bfc)


def kernel(x, edge_index, W1, b1, W2, b2, Wfc, bfc):
    src = edge_index[0].astype(jnp.int32)
    dst = edge_index[1].astype(jnp.int32)
    pad = jnp.arange(EP - E, dtype=jnp.int32)
    srcp = jnp.concatenate([src, pad % N]).reshape(EP // CHUNK, CHUNK)
    dstp = jnp.concatenate([dst, N + pad % (NP - N)]).reshape(EP // CHUNK, CHUNK)

    ones16 = jnp.ones((CHUNK, 16), _f32)
    z16 = jnp.zeros((ROWS_PT, 16), _f32)
    z64 = jnp.zeros((ROWS_PT, 64), _f32)
    z32 = jnp.zeros((ROWS_PT, 32), _f32)

    dp = _deg_sc(dstp, ones16, z16)
    y1, s1 = _dense1(x, W1, dp)
    a1 = _agg64(y1, srcp, dstp, z64)
    y2, s2 = _dense2(a1, s1, dp, W2, b1.reshape(1, 64))
    a2 = _agg32(y2, srcp, dstp, z32)
    return _dense3(a2, s2, dp, Wfc.reshape(1, 32),
                   b2.reshape(1, 32), bfc.reshape(1, 1))


# speedup vs baseline: 1.1671x; 1.1671x over previous
import functools

import jax
import jax.numpy as jnp
from jax import lax
from jax.experimental import pallas as pl
from jax.experimental.pallas import tpu as pltpu
from jax.experimental.pallas import tpu_sc as plsc

N = 10000
E = 160000
NP = 10240
EP = 163840
CHUNK = 128
CPT = (EP // 32) // CHUNK

_mesh = plsc.VectorSubcoreMesh(core_axis_name="c", subcore_axis_name="s")
_f32 = jnp.float32
_sc_params = pltpu.CompilerParams(use_tc_tiling_on_sc=False)


@functools.partial(
    pl.kernel,
    out_type=jax.ShapeDtypeStruct((2, NP, 64), _f32),
    mesh=_mesh,
    scratch_types=[
        pltpu.VMEM((CPT, CHUNK), jnp.int32),
        pltpu.VMEM((CHUNK, 64), _f32),
        pltpu.VMEM_SHARED((NP, 64), _f32),
        pltpu.SemaphoreType.DMA,
    ],
    compiler_params=_sc_params,
)
def _deg_sc(dst_hbm, ones_hbm, zeros_hbm, out_hbm, idx_v, ones_v, acc, sem):
    cid = lax.axis_index("c")
    sid = lax.axis_index("s")
    tid = cid * 16 + sid
    rpt = NP // 16
    pltpu.sync_copy(dst_hbm.at[pl.ds(tid * CPT, CPT)], idx_v)
    pltpu.sync_copy(ones_hbm, ones_v)
    pltpu.sync_copy(zeros_hbm, acc.at[pl.ds(sid * rpt, rpt)])
    plsc.subcore_barrier()

    @pl.loop(0, CPT, step=8)
    def _(j):
        for k in range(8):
            pltpu.async_copy(ones_v, acc.at[idx_v.at[j + k]], sem, add=True)
        for k in range(8):
            pltpu.make_async_copy(ones_v, acc.at[idx_v.at[j + k]], sem).wait()

    plsc.subcore_barrier()
    pltpu.sync_copy(
        acc.at[pl.ds(sid * rpt, rpt)],
        out_hbm.at[cid, pl.ds(sid * rpt, rpt)],
    )


def _make_agg(nrows, d):
    rpt = nrows // 16

    @functools.partial(
        pl.kernel,
        out_type=jax.ShapeDtypeStruct((2, nrows, d), _f32),
        mesh=_mesh,
        scratch_types=[
            pltpu.VMEM((CPT, CHUNK), jnp.int32),
            pltpu.VMEM((CPT, CHUNK), jnp.int32),
            [pltpu.VMEM((CHUNK, d), _f32)] * 8,
            [pltpu.SemaphoreType.DMA] * 8,
            [pltpu.SemaphoreType.DMA] * 8,
            pltpu.VMEM_SHARED((nrows, d), _f32),
        ],
        compiler_params=_sc_params,
    )
    def _agg(y_hbm, src_hbm, dst_hbm, zeros_hbm, out_hbm,
             srcv, dstv, bufs, gsems, ssems, acc):
        cid = lax.axis_index("c")
        sid = lax.axis_index("s")
        tid = cid * 16 + sid
        pltpu.sync_copy(src_hbm.at[pl.ds(tid * CPT, CPT)], srcv)
        pltpu.sync_copy(dst_hbm.at[pl.ds(tid * CPT, CPT)], dstv)
        pltpu.sync_copy(zeros_hbm, acc.at[pl.ds(sid * rpt, rpt)])
        plsc.subcore_barrier()

        for c in range(4):
            pltpu.async_copy(y_hbm.at[srcv.at[c]], bufs[c], gsems[c])

        @pl.loop(0, CPT, step=8)
        def _(j):
            for k in range(8):
                b = k % 8
                pltpu.make_async_copy(y_hbm.at[srcv.at[j + k]],
                                      bufs[b], gsems[b]).wait()
                pltpu.async_copy(bufs[b], acc.at[dstv.at[j + k]],
                                 ssems[b], add=True)
                bn = (k + 4) % 8

                @pl.when(j + k + 4 < CPT)
                def _():
                    @pl.when(j + k >= 4)
                    def _():
                        pltpu.make_async_copy(
                            bufs[bn], acc.at[dstv.at[j + k - 4]],
                            ssems[bn]).wait()

                    pltpu.async_copy(y_hbm.at[srcv.at[j + k + 4]],
                                     bufs[bn], gsems[bn])

        for c in range(CPT - 8, CPT):
            b = c % 8
            pltpu.make_async_copy(bufs[b], acc.at[dstv.at[c]],
                                  ssems[b]).wait()

        plsc.subcore_barrier()
        pltpu.sync_copy(
            acc.at[pl.ds(sid * rpt, rpt)],
            out_hbm.at[cid, pl.ds(sid * rpt, rpt)],
        )

    return _agg


_agg64 = _make_agg(NP, 64)
_agg32 = _make_agg(2 * NP, 32)


_R = 1024
_GRID = (NP // 2) // _R


def _disf(dpf_ref):
    degf = 1.0 + dpf_ref[0] + dpf_ref[1]
    return 1.0 / jnp.sqrt(degf)


def _dense1_body(xf_ref, wbd_ref, dpf_ref, y_ref, s_ref):
    xwf = jnp.dot(xf_ref[...], wbd_ref[...], preferred_element_type=_f32)
    disf = _disf(dpf_ref)
    y_ref[...] = xwf * disf
    s_ref[...] = xwf * (disf * disf)


def _dense1(xf, w1bd, dpf):
    return pl.pallas_call(
        _dense1_body,
        grid=(_GRID,),
        in_specs=[
            pl.BlockSpec((_R, 512), lambda i: (i, 0)),
            pl.BlockSpec((512, 128), lambda i: (0, 0)),
            pl.BlockSpec((2, _R, 128), lambda i: (0, i, 0)),
        ],
        out_specs=[
            pl.BlockSpec((_R, 128), lambda i: (i, 0)),
            pl.BlockSpec((_R, 128), lambda i: (i, 0)),
        ],
        out_shape=[
            jax.ShapeDtypeStruct((NP // 2, 128), _f32),
            jax.ShapeDtypeStruct((NP // 2, 128), _f32),
        ],
    )(xf, w1bd, dpf)


def _dense2_body(af_ref, sf_ref, dpf_ref, wbd_ref, bf_ref, yz_ref):
    disf = _disf(dpf_ref)
    aggf = af_ref[0] + af_ref[1]
    hf = jnp.maximum(aggf * disf + sf_ref[...] + bf_ref[...], 0.0)
    xw2 = jnp.dot(hf, wbd_ref[...], preferred_element_type=_f32)
    dis32 = jnp.concatenate([disf[:, 0:32], disf[:, 64:96]], axis=1)
    y2 = xw2 * dis32
    s2 = xw2 * (dis32 * dis32)
    yz_ref[...] = jnp.concatenate([y2, s2], axis=1)


def _dense2(af, sf, dpf, w2bd, b1f):
    return pl.pallas_call(
        _dense2_body,
        grid=(_GRID,),
        in_specs=[
            pl.BlockSpec((2, _R, 128), lambda i: (0, i, 0)),
            pl.BlockSpec((_R, 128), lambda i: (i, 0)),
            pl.BlockSpec((2, _R, 128), lambda i: (0, i, 0)),
            pl.BlockSpec((128, 64), lambda i: (0, 0)),
            pl.BlockSpec((1, 128), lambda i: (0, 0)),
        ],
        out_specs=pl.BlockSpec((_R, 128), lambda i: (i, 0)),
        out_shape=jax.ShapeDtypeStruct((NP // 2, 128), _f32),
    )(af, sf, dpf, w2bd, b1f)


def _dense3_body(az_ref, yz_ref, dpf_ref, wbd_ref, bf_ref, bfc_ref, o_ref):
    disf = _disf(dpf_ref)
    dis32 = jnp.concatenate([disf[:, 0:32], disf[:, 64:96]], axis=1)
    azs = az_ref[0] + az_ref[1]
    a2 = azs[:, 0:64]
    s2 = yz_ref[...][:, 64:128]
    hf = jnp.maximum(a2 * dis32 + s2 + bf_ref[...], 0.0)
    o_ref[...] = (jnp.dot(hf, wbd_ref[...], preferred_element_type=_f32)
                  + bfc_ref[...])


def _dense3(az, yz, dpf, wfcbd, b2f, bfc):
    return pl.pallas_call(
        _dense3_body,
        grid=(_GRID,),
        in_specs=[
            pl.BlockSpec((2, _R, 128), lambda i: (0, i, 0)),
            pl.BlockSpec((_R, 128), lambda i: (i, 0)),
            pl.BlockSpec((2, _R, 128), lambda i: (0, i, 0)),
            pl.BlockSpec((64, 2), lambda i: (0, 0)),
            pl.BlockSpec((1, 64), lambda i: (0, 0)),
            pl.BlockSpec((1, 1), lambda i: (0, 0)),
        ],
        out_specs=pl.BlockSpec((_R, 2), lambda i: (i, 0)),
        out_shape=jax.ShapeDtypeStruct((NP // 2, 2), _f32),
    )(az, yz, dpf, wfcbd, b2f, bfc)


def kernel(x, edge_index, W1, b1, W2, b2, Wfc, bfc):
    src = edge_index[0].astype(jnp.int32)
    dst = edge_index[1].astype(jnp.int32)
    pad = jnp.arange(EP - E, dtype=jnp.int32)
    srcp = jnp.concatenate([src, pad % N])
    dstp = jnp.concatenate([dst, N + pad % (NP - N)])
    srcp2 = srcp.reshape(EP // CHUNK, CHUNK)
    dstp2 = dstp.reshape(EP // CHUNK, CHUNK)
    psi_s = (4 * (srcp // 2) + (srcp % 2)).reshape(EP // CHUNK, CHUNK)
    psi_d = (4 * (dstp // 2) + (dstp % 2)).reshape(EP // CHUNK, CHUNK)

    ones64 = jnp.ones((CHUNK, 64), _f32)
    z64 = jnp.zeros((NP // 16, 64), _f32)
    z32 = jnp.zeros((2 * NP // 16, 32), _f32)

    w1bd = jnp.zeros((512, 128), _f32).at[:256, :64].set(W1).at[256:, 64:].set(W1)
    w2bd = jnp.zeros((128, 64), _f32).at[:64, :32].set(W2).at[64:, 32:].set(W2)
    wfcbd = jnp.zeros((64, 2), _f32).at[:32, 0].set(Wfc[:, 0]).at[32:, 1].set(Wfc[:, 0])
    b1f = jnp.tile(b1, 2).reshape(1, 128)
    b2f = jnp.tile(b2, 2).reshape(1, 64)

    xf = x.reshape(N // 2, 512)

    dp = _deg_sc(dstp2, ones64, z64)
    dpf = dp.reshape(2, NP // 2, 128)
    y1f, s1f = _dense1(xf, w1bd, dpf)
    a1 = _agg64(y1f.reshape(NP, 64), srcp2, dstp2, z64)
    yz = _dense2(a1.reshape(2, NP // 2, 128), s1f, dpf, w2bd, b1f)
    a2 = _agg32(yz.reshape(2 * NP, 32), psi_s, psi_d, z32)
    o = _dense3(a2.reshape(2, NP // 2, 128), yz, dpf, wfcbd, b2f,
                bfc.reshape(1, 1))
    return o.reshape(NP, 1)[:N]


# speedup vs baseline: 1.2101x; 1.0369x over previous
import functools

import jax
import jax.numpy as jnp
from jax import lax
from jax.experimental import pallas as pl
from jax.experimental.pallas import tpu as pltpu
from jax.experimental.pallas import tpu_sc as plsc

N = 10000
E = 160000
NP = 10240
EP = 163840
CHUNK = 128
CPT = (EP // 32) // CHUNK

_mesh = plsc.VectorSubcoreMesh(core_axis_name="c", subcore_axis_name="s")
_f32 = jnp.float32
_sc_params = pltpu.CompilerParams(use_tc_tiling_on_sc=False)


@functools.partial(
    pl.kernel,
    out_type=jax.ShapeDtypeStruct((2, NP, 64), _f32),
    mesh=_mesh,
    scratch_types=[
        pltpu.VMEM((CPT, CHUNK), jnp.int32),
        pltpu.VMEM((CHUNK, 64), _f32),
        pltpu.VMEM_SHARED((NP, 64), _f32),
        pltpu.SemaphoreType.DMA,
    ],
    compiler_params=_sc_params,
)
def _deg_sc(dst_hbm, ones_hbm, zeros_hbm, out_hbm, idx_v, ones_v, acc, sem):
    cid = lax.axis_index("c")
    sid = lax.axis_index("s")
    tid = cid * 16 + sid
    rpt = NP // 16
    pltpu.sync_copy(dst_hbm.at[pl.ds(tid * CPT, CPT)], idx_v)
    pltpu.sync_copy(ones_hbm, ones_v)
    pltpu.sync_copy(zeros_hbm, acc.at[pl.ds(sid * rpt, rpt)])
    plsc.subcore_barrier()

    @pl.loop(0, CPT, step=8)
    def _(j):
        for k in range(8):
            pltpu.async_copy(ones_v, acc.at[idx_v.at[j + k]], sem, add=True)
        for k in range(8):
            pltpu.make_async_copy(ones_v, acc.at[idx_v.at[j + k]], sem).wait()

    plsc.subcore_barrier()
    pltpu.sync_copy(
        acc.at[pl.ds(sid * rpt, rpt)],
        out_hbm.at[cid, pl.ds(sid * rpt, rpt)],
    )


def _make_agg(nrows, d):
    rpt = nrows // 16

    @functools.partial(
        pl.kernel,
        out_type=jax.ShapeDtypeStruct((2, nrows, d), _f32),
        mesh=_mesh,
        scratch_types=[
            pltpu.VMEM((CPT, CHUNK), jnp.int32),
            pltpu.VMEM((CPT, CHUNK), jnp.int32),
            [pltpu.VMEM((CHUNK, d), _f32)] * 8,
            [pltpu.SemaphoreType.DMA] * 8,
            [pltpu.SemaphoreType.DMA] * 8,
            pltpu.VMEM_SHARED((nrows, d), _f32),
        ],
        compiler_params=_sc_params,
    )
    def _agg(y_hbm, src_hbm, dst_hbm, zeros_hbm, out_hbm,
             srcv, dstv, bufs, gsems, ssems, acc):
        cid = lax.axis_index("c")
        sid = lax.axis_index("s")
        tid = cid * 16 + sid
        pltpu.sync_copy(src_hbm.at[pl.ds(tid * CPT, CPT)], srcv)
        pltpu.sync_copy(dst_hbm.at[pl.ds(tid * CPT, CPT)], dstv)
        pltpu.sync_copy(zeros_hbm, acc.at[pl.ds(sid * rpt, rpt)])
        plsc.subcore_barrier()

        for c in range(4):
            pltpu.async_copy(y_hbm.at[srcv.at[c]], bufs[c], gsems[c])

        @pl.loop(0, CPT, step=8)
        def _(j):
            for k in range(8):
                b = k % 8
                pltpu.make_async_copy(y_hbm.at[srcv.at[j + k]],
                                      bufs[b], gsems[b]).wait()
                pltpu.async_copy(bufs[b], acc.at[dstv.at[j + k]],
                                 ssems[b], add=True)
                bn = (k + 4) % 8

                @pl.when(j + k + 4 < CPT)
                def _():
                    @pl.when(j + k >= 4)
                    def _():
                        pltpu.make_async_copy(
                            bufs[bn], acc.at[dstv.at[j + k - 4]],
                            ssems[bn]).wait()

                    pltpu.async_copy(y_hbm.at[srcv.at[j + k + 4]],
                                     bufs[bn], gsems[bn])

        for c in range(CPT - 8, CPT):
            b = c % 8
            pltpu.make_async_copy(bufs[b], acc.at[dstv.at[c]],
                                  ssems[b]).wait()

        plsc.subcore_barrier()
        pltpu.sync_copy(
            acc.at[pl.ds(sid * rpt, rpt)],
            out_hbm.at[cid, pl.ds(sid * rpt, rpt)],
        )

    return _agg


_agg64 = _make_agg(NP, 64)
_agg32 = _make_agg(NP, 32)


_R = 1024
_GRID = (NP // 2) // _R


def _disf(dpf_ref):
    degf = 1.0 + dpf_ref[0] + dpf_ref[1]
    return 1.0 / jnp.sqrt(degf)


def _dense1_body(xf_ref, wbd_ref, dpf_ref, y_ref, s_ref):
    xwf = jnp.dot(xf_ref[...], wbd_ref[...], preferred_element_type=_f32)
    disf = _disf(dpf_ref)
    y_ref[...] = xwf * disf
    s_ref[...] = xwf * (disf * disf)


def _dense1(xf, w1bd, dpf):
    return pl.pallas_call(
        _dense1_body,
        grid=(_GRID,),
        in_specs=[
            pl.BlockSpec((_R, 512), lambda i: (i, 0)),
            pl.BlockSpec((512, 128), lambda i: (0, 0)),
            pl.BlockSpec((2, _R, 128), lambda i: (0, i, 0)),
        ],
        out_specs=[
            pl.BlockSpec((_R, 128), lambda i: (i, 0)),
            pl.BlockSpec((_R, 128), lambda i: (i, 0)),
        ],
        out_shape=[
            jax.ShapeDtypeStruct((NP // 2, 128), _f32),
            jax.ShapeDtypeStruct((NP // 2, 128), _f32),
        ],
    )(xf, w1bd, dpf)


def _dense2_body(af_ref, sf_ref, dpf_ref, wbd_ref, bf_ref, yz_ref):
    disf = _disf(dpf_ref)
    aggf = af_ref[0] + af_ref[1]
    hf = jnp.maximum(aggf * disf + sf_ref[...] + bf_ref[...], 0.0)
    xw2 = jnp.dot(hf, wbd_ref[...], preferred_element_type=_f32)
    dis32 = jnp.concatenate([disf[:, 0:32], disf[:, 64:96]], axis=1)
    y2 = xw2 * dis32
    s2 = xw2 * (dis32 * dis32)
    yz_ref[...] = jnp.concatenate([y2, s2], axis=1)


def _dense2(af, sf, dpf, w2bd, b1f):
    return pl.pallas_call(
        _dense2_body,
        grid=(_GRID,),
        in_specs=[
            pl.BlockSpec((2, _R, 128), lambda i: (0, i, 0)),
            pl.BlockSpec((_R, 128), lambda i: (i, 0)),
            pl.BlockSpec((2, _R, 128), lambda i: (0, i, 0)),
            pl.BlockSpec((128, 64), lambda i: (0, 0)),
            pl.BlockSpec((1, 128), lambda i: (0, 0)),
        ],
        out_specs=pl.BlockSpec((_R, 128), lambda i: (i, 0)),
        out_shape=jax.ShapeDtypeStruct((NP // 2, 128), _f32),
    )(af, sf, dpf, w2bd, b1f)


def _dense3_body(af_ref, yz4_ref, dp4_ref, wbd_ref, bf_ref, bfc_ref, o_ref):
    deg4 = 1.0 + dp4_ref[0] + dp4_ref[1]
    dis4 = 1.0 / jnp.sqrt(deg4)
    dis32 = jnp.concatenate(
        [dis4[:, 0:32], dis4[:, 64:96], dis4[:, 128:160], dis4[:, 192:224]],
        axis=1)
    yz4 = yz4_ref[...]
    s2 = jnp.concatenate([yz4[:, 64:128], yz4[:, 192:256]], axis=1)
    a2 = af_ref[0] + af_ref[1]
    hf = jnp.maximum(a2 * dis32 + s2 + bf_ref[...], 0.0)
    o_ref[...] = (jnp.dot(hf, wbd_ref[...], preferred_element_type=_f32)
                  + bfc_ref[...])


def _dense3(af, yz4, dp4, wfcbd, b2f, bfc):
    return pl.pallas_call(
        _dense3_body,
        grid=(_GRID,),
        in_specs=[
            pl.BlockSpec((2, _R // 2, 128), lambda i: (0, i, 0)),
            pl.BlockSpec((_R // 2, 256), lambda i: (i, 0)),
            pl.BlockSpec((2, _R // 2, 256), lambda i: (0, i, 0)),
            pl.BlockSpec((128, 4), lambda i: (0, 0)),
            pl.BlockSpec((1, 128), lambda i: (0, 0)),
            pl.BlockSpec((1, 1), lambda i: (0, 0)),
        ],
        out_specs=pl.BlockSpec((_R // 2, 4), lambda i: (i, 0)),
        out_shape=jax.ShapeDtypeStruct((NP // 4, 4), _f32),
    )(af, yz4, dp4, wfcbd, b2f, bfc)


def kernel(x, edge_index, W1, b1, W2, b2, Wfc, bfc):
    src = edge_index[0].astype(jnp.int32)
    dst = edge_index[1].astype(jnp.int32)
    pad = jnp.arange(EP - E, dtype=jnp.int32)
    srcp = jnp.concatenate([src, pad % N])
    dstp = jnp.concatenate([dst, N + pad % (NP - N)])
    srcp2 = srcp.reshape(EP // CHUNK, CHUNK)
    dstp2 = dstp.reshape(EP // CHUNK, CHUNK)
    psi_s = (4 * (srcp // 2) + (srcp % 2)).reshape(EP // CHUNK, CHUNK)

    ones64 = jnp.ones((CHUNK, 64), _f32)
    z64 = jnp.zeros((NP // 16, 64), _f32)
    z32 = jnp.zeros((NP // 16, 32), _f32)

    w1bd = jnp.zeros((512, 128), _f32).at[:256, :64].set(W1).at[256:, 64:].set(W1)
    w2bd = jnp.zeros((128, 64), _f32).at[:64, :32].set(W2).at[64:, 32:].set(W2)
    wfcbd = jnp.zeros((128, 4), _f32)
    for k in range(4):
        wfcbd = wfcbd.at[32 * k:32 * (k + 1), k].set(Wfc[:, 0])
    b1f = jnp.tile(b1, 2).reshape(1, 128)
    b2f = jnp.tile(b2, 4).reshape(1, 128)

    xf = x.reshape(N // 2, 512)

    dp = _deg_sc(dstp2, ones64, z64)
    dpf = dp.reshape(2, NP // 2, 128)
    y1f, s1f = _dense1(xf, w1bd, dpf)
    a1 = _agg64(y1f.reshape(NP, 64), srcp2, dstp2, z64)
    yz = _dense2(a1.reshape(2, NP // 2, 128), s1f, dpf, w2bd, b1f)
    a2 = _agg32(yz.reshape(2 * NP, 32), psi_s, dstp2, z32)
    o = _dense3(a2.reshape(2, NP // 4, 128), yz.reshape(NP // 4, 256),
                dp.reshape(2, NP // 4, 256), wfcbd, b2f,
                bfc.reshape(1, 1))
    return o.reshape(NP, 1)[:N]


# speedup vs baseline: 1.3099x; 1.0824x over previous
import functools

import jax
import jax.numpy as jnp
from jax import lax
from jax.experimental import pallas as pl
from jax.experimental.pallas import tpu as pltpu
from jax.experimental.pallas import tpu_sc as plsc

N = 10000
E = 160000
NP = 10240
EP = 163840
CHUNK = 128
CPT = (EP // 32) // CHUNK

_mesh = plsc.VectorSubcoreMesh(core_axis_name="c", subcore_axis_name="s")
_f32 = jnp.float32
_sc_params = pltpu.CompilerParams(use_tc_tiling_on_sc=False)
_sc_vec_params = pltpu.CompilerParams(use_tc_tiling_on_sc=False,
                                      needs_layout_passes=False)
_RPT = NP // 16


@functools.partial(
    pl.kernel,
    out_type=jax.ShapeDtypeStruct((2, NP, 64), _f32),
    mesh=_mesh,
    scratch_types=[
        pltpu.VMEM((CPT, CHUNK), jnp.int32),
        pltpu.VMEM((NP,), _f32),
        pltpu.VMEM((16, _RPT), _f32),
        pltpu.VMEM((_RPT,), _f32),
        pltpu.VMEM((_RPT, 64), _f32),
        pltpu.VMEM_SHARED((16, NP), _f32),
        pltpu.SemaphoreType.DMA,
    ],
    compiler_params=_sc_vec_params,
)
def _deg_sc(dst_hbm, znp_hbm, out_hbm, idx_v, hist, mbuf, sums, obuf,
            shared, sem):
    cid = lax.axis_index("c")
    sid = lax.axis_index("s")
    tid = cid * 16 + sid
    pltpu.sync_copy(dst_hbm.at[pl.ds(tid * CPT, CPT)], idx_v)
    pltpu.sync_copy(znp_hbm, hist)
    ones = jnp.ones((16,), _f32)

    @pl.loop(0, CPT)
    def _(j):
        for l in range(8):
            idx16 = idx_v[j, pl.ds(16 * l, 16)]
            plsc.addupdate_scatter(hist, [idx16], ones)

    pltpu.sync_copy(hist, shared.at[sid])
    plsc.subcore_barrier()
    pltpu.sync_copy(shared.at[:, pl.ds(sid * _RPT, _RPT)], mbuf)

    @pl.loop(0, _RPT, step=16)
    def _(i):
        v = mbuf[0, pl.ds(i, 16)]
        for t in range(1, 16):
            v = v + mbuf[t, pl.ds(i, 16)]
        sums[pl.ds(i, 16)] = v

    @pl.loop(0, _RPT)
    def _(n):
        row = plsc.load_gather(sums, [jnp.full((16,), n, jnp.int32)])
        for k in range(4):
            obuf[n, pl.ds(16 * k, 16)] = row

    pltpu.sync_copy(obuf, out_hbm.at[cid, pl.ds(sid * _RPT, _RPT)])


def _make_agg(nrows, d):
    rpt = nrows // 16

    @functools.partial(
        pl.kernel,
        out_type=jax.ShapeDtypeStruct((2, nrows, d), _f32),
        mesh=_mesh,
        scratch_types=[
            pltpu.VMEM((CPT, CHUNK), jnp.int32),
            pltpu.VMEM((CPT, CHUNK), jnp.int32),
            [pltpu.VMEM((CHUNK, d), _f32)] * 8,
            [pltpu.SemaphoreType.DMA] * 8,
            [pltpu.SemaphoreType.DMA] * 8,
            pltpu.VMEM_SHARED((nrows, d), _f32),
        ],
        compiler_params=_sc_params,
    )
    def _agg(y_hbm, src_hbm, dst_hbm, zeros_hbm, out_hbm,
             srcv, dstv, bufs, gsems, ssems, acc):
        cid = lax.axis_index("c")
        sid = lax.axis_index("s")
        tid = cid * 16 + sid
        pltpu.sync_copy(src_hbm.at[pl.ds(tid * CPT, CPT)], srcv)
        pltpu.sync_copy(dst_hbm.at[pl.ds(tid * CPT, CPT)], dstv)
        pltpu.sync_copy(zeros_hbm, acc.at[pl.ds(sid * rpt, rpt)])
        plsc.subcore_barrier()

        for c in range(4):
            pltpu.async_copy(y_hbm.at[srcv.at[c]], bufs[c], gsems[c])

        @pl.loop(0, CPT, step=8)
        def _(j):
            for k in range(8):
                b = k % 8
                pltpu.make_async_copy(y_hbm.at[srcv.at[j + k]],
                                      bufs[b], gsems[b]).wait()
                pltpu.async_copy(bufs[b], acc.at[dstv.at[j + k]],
                                 ssems[b], add=True)
                bn = (k + 4) % 8

                @pl.when(j + k + 4 < CPT)
                def _():
                    @pl.when(j + k >= 4)
                    def _():
                        pltpu.make_async_copy(
                            bufs[bn], acc.at[dstv.at[j + k - 4]],
                            ssems[bn]).wait()

                    pltpu.async_copy(y_hbm.at[srcv.at[j + k + 4]],
                                     bufs[bn], gsems[bn])

        for c in range(CPT - 8, CPT):
            b = c % 8
            pltpu.make_async_copy(bufs[b], acc.at[dstv.at[c]],
                                  ssems[b]).wait()

        plsc.subcore_barrier()
        pltpu.sync_copy(
            acc.at[pl.ds(sid * rpt, rpt)],
            out_hbm.at[cid, pl.ds(sid * rpt, rpt)],
        )

    return _agg


_agg64 = _make_agg(NP, 64)
_agg32 = _make_agg(NP, 32)


_R = 1024
_GRID = (NP // 2) // _R


def _disf(dpf_ref):
    degf = 1.0 + dpf_ref[0] + dpf_ref[1]
    return 1.0 / jnp.sqrt(degf)


def _dense1_body(xf_ref, wbd_ref, dpf_ref, y_ref, s_ref):
    xwf = jnp.dot(xf_ref[...], wbd_ref[...], preferred_element_type=_f32)
    disf = _disf(dpf_ref)
    y_ref[...] = xwf * disf
    s_ref[...] = xwf * (disf * disf)


def _dense1(xf, w1bd, dpf):
    return pl.pallas_call(
        _dense1_body,
        grid=(_GRID,),
        in_specs=[
            pl.BlockSpec((_R, 512), lambda i: (i, 0)),
            pl.BlockSpec((512, 128), lambda i: (0, 0)),
            pl.BlockSpec((2, _R, 128), lambda i: (0, i, 0)),
        ],
        out_specs=[
            pl.BlockSpec((_R, 128), lambda i: (i, 0)),
            pl.BlockSpec((_R, 128), lambda i: (i, 0)),
        ],
        out_shape=[
            jax.ShapeDtypeStruct((NP // 2, 128), _f32),
            jax.ShapeDtypeStruct((NP // 2, 128), _f32),
        ],
    )(xf, w1bd, dpf)


def _dense2_body(af_ref, sf_ref, dpf_ref, wbd_ref, bf_ref, yz_ref):
    disf = _disf(dpf_ref)
    aggf = af_ref[0] + af_ref[1]
    hf = jnp.maximum(aggf * disf + sf_ref[...] + bf_ref[...], 0.0)
    xw2 = jnp.dot(hf, wbd_ref[...], preferred_element_type=_f32)
    dis32 = jnp.concatenate([disf[:, 0:32], disf[:, 64:96]], axis=1)
    y2 = xw2 * dis32
    s2 = xw2 * (dis32 * dis32)
    yz_ref[...] = jnp.concatenate([y2, s2], axis=1)


def _dense2(af, sf, dpf, w2bd, b1f):
    return pl.pallas_call(
        _dense2_body,
        grid=(_GRID,),
        in_specs=[
            pl.BlockSpec((2, _R, 128), lambda i: (0, i, 0)),
            pl.BlockSpec((_R, 128), lambda i: (i, 0)),
            pl.BlockSpec((2, _R, 128), lambda i: (0, i, 0)),
            pl.BlockSpec((128, 64), lambda i: (0, 0)),
            pl.BlockSpec((1, 128), lambda i: (0, 0)),
        ],
        out_specs=pl.BlockSpec((_R, 128), lambda i: (i, 0)),
        out_shape=jax.ShapeDtypeStruct((NP // 2, 128), _f32),
    )(af, sf, dpf, w2bd, b1f)


def _dense3_body(af_ref, yz4_ref, dp4_ref, wbd_ref, bf_ref, bfc_ref, o_ref):
    deg4 = 1.0 + dp4_ref[0] + dp4_ref[1]
    dis4 = 1.0 / jnp.sqrt(deg4)
    dis32 = jnp.concatenate(
        [dis4[:, 0:32], dis4[:, 64:96], dis4[:, 128:160], dis4[:, 192:224]],
        axis=1)
    yz4 = yz4_ref[...]
    s2 = jnp.concatenate([yz4[:, 64:128], yz4[:, 192:256]], axis=1)
    a2 = af_ref[0] + af_ref[1]
    hf = jnp.maximum(a2 * dis32 + s2 + bf_ref[...], 0.0)
    o_ref[...] = (jnp.dot(hf, wbd_ref[...], preferred_element_type=_f32)
                  + bfc_ref[...])


def _dense3(af, yz4, dp4, wfcbd, b2f, bfc):
    return pl.pallas_call(
        _dense3_body,
        grid=(_GRID,),
        in_specs=[
            pl.BlockSpec((2, _R // 2, 128), lambda i: (0, i, 0)),
            pl.BlockSpec((_R // 2, 256), lambda i: (i, 0)),
            pl.BlockSpec((2, _R // 2, 256), lambda i: (0, i, 0)),
            pl.BlockSpec((128, 4), lambda i: (0, 0)),
            pl.BlockSpec((1, 128), lambda i: (0, 0)),
            pl.BlockSpec((1, 1), lambda i: (0, 0)),
        ],
        out_specs=pl.BlockSpec((_R // 2, 4), lambda i: (i, 0)),
        out_shape=jax.ShapeDtypeStruct((NP // 4, 4), _f32),
    )(af, yz4, dp4, wfcbd, b2f, bfc)


def kernel(x, edge_index, W1, b1, W2, b2, Wfc, bfc):
    src = edge_index[0].astype(jnp.int32)
    dst = edge_index[1].astype(jnp.int32)
    pad = jnp.arange(EP - E, dtype=jnp.int32)
    srcp = jnp.concatenate([src, pad % N])
    dstp = jnp.concatenate([dst, N + pad % (NP - N)])
    srcp2 = srcp.reshape(EP // CHUNK, CHUNK)
    dstp2 = dstp.reshape(EP // CHUNK, CHUNK)
    psi_s = (4 * (srcp // 2) + (srcp % 2)).reshape(EP // CHUNK, CHUNK)

    znp = jnp.zeros((NP,), _f32)
    z64 = jnp.zeros((NP // 16, 64), _f32)
    z32 = jnp.zeros((NP // 16, 32), _f32)

    w1bd = jnp.zeros((512, 128), _f32).at[:256, :64].set(W1).at[256:, 64:].set(W1)
    w2bd = jnp.zeros((128, 64), _f32).at[:64, :32].set(W2).at[64:, 32:].set(W2)
    wfcbd = jnp.zeros((128, 4), _f32)
    for k in range(4):
        wfcbd = wfcbd.at[32 * k:32 * (k + 1), k].set(Wfc[:, 0])
    b1f = jnp.tile(b1, 2).reshape(1, 128)
    b2f = jnp.tile(b2, 4).reshape(1, 128)

    xf = x.reshape(N // 2, 512)

    dp = _deg_sc(dstp2, znp)
    dpf = dp.reshape(2, NP // 2, 128)
    y1f, s1f = _dense1(xf, w1bd, dpf)
    a1 = _agg64(y1f.reshape(NP, 64), srcp2, dstp2, z64)
    yz = _dense2(a1.reshape(2, NP // 2, 128), s1f, dpf, w2bd, b1f)
    a2 = _agg32(yz.reshape(2 * NP, 32), psi_s, dstp2, z32)
    o = _dense3(a2.reshape(2, NP // 4, 128), yz.reshape(NP // 4, 256),
                dp.reshape(2, NP // 4, 256), wfcbd, b2f,
                bfc.reshape(1, 1))
    return o.reshape(NP, 1)[:N]


# speedup vs baseline: 1.3874x; 1.0592x over previous
import functools

import jax
import jax.numpy as jnp
import numpy as np
from jax import lax
from jax.experimental import pallas as pl
from jax.experimental.pallas import tpu as pltpu
from jax.experimental.pallas import tpu_sc as plsc

N = 10000
E = 160000
NP = 10240
EP = 163840
CHUNK = 128
CPT = (EP // 32) // CHUNK
ECH = E // CHUNK
_PAD_NP = np.stack([
    np.arange(30 * CHUNK).reshape(30, CHUNK) % N,
    N + np.arange(30 * CHUNK).reshape(30, CHUNK) % (NP - N),
], axis=1).astype(np.int32)

_mesh = plsc.VectorSubcoreMesh(core_axis_name="c", subcore_axis_name="s")
_f32 = jnp.float32
_sc_params = pltpu.CompilerParams(use_tc_tiling_on_sc=False)
_sc_vec_params = pltpu.CompilerParams(use_tc_tiling_on_sc=False,
                                      needs_layout_passes=False)
_RPT = NP // 16


def _stage_edges(ei_hbm, pad_hbm, eiv, tid):
    @pl.when(tid < 31)
    def _():
        pltpu.sync_copy(ei_hbm.at[pl.ds(tid * CPT, CPT)], eiv)

    @pl.when(tid == 31)
    def _():
        pltpu.sync_copy(ei_hbm.at[pl.ds(ECH - 10, 10)], eiv.at[pl.ds(0, 10)])
        pltpu.sync_copy(pad_hbm, eiv.at[pl.ds(10, 30)])


@functools.partial(
    pl.kernel,
    out_type=jax.ShapeDtypeStruct((2, NP, 64), _f32),
    mesh=_mesh,
    scratch_types=[
        pltpu.VMEM((CPT, 2, CHUNK), jnp.int32),
        pltpu.VMEM((NP,), _f32),
        pltpu.VMEM((16, _RPT), _f32),
        pltpu.VMEM((_RPT,), _f32),
        pltpu.VMEM((_RPT, 64), _f32),
        pltpu.VMEM_SHARED((16, NP), _f32),
        pltpu.SemaphoreType.DMA,
    ],
    compiler_params=_sc_vec_params,
)
def _deg_sc(ei_hbm, pad_hbm, znp_hbm, out_hbm, eiv, hist, mbuf, sums, obuf,
            shared, sem):
    cid = lax.axis_index("c")
    sid = lax.axis_index("s")
    tid = cid * 16 + sid
    _stage_edges(ei_hbm, pad_hbm, eiv, tid)
    pltpu.sync_copy(znp_hbm, hist)
    ones = jnp.ones((16,), _f32)

    @pl.loop(0, CPT)
    def _(j):
        for l in range(8):
            idx16 = eiv[j, 1, pl.ds(16 * l, 16)]
            plsc.addupdate_scatter(hist, [idx16], ones)

    pltpu.sync_copy(hist, shared.at[sid])
    plsc.subcore_barrier()
    pltpu.sync_copy(shared.at[:, pl.ds(sid * _RPT, _RPT)], mbuf)

    @pl.loop(0, _RPT, step=16)
    def _(i):
        v = mbuf[0, pl.ds(i, 16)]
        for t in range(1, 16):
            v = v + mbuf[t, pl.ds(i, 16)]
        sums[pl.ds(i, 16)] = v

    @pl.loop(0, _RPT)
    def _(n):
        row = plsc.load_gather(sums, [jnp.full((16,), n, jnp.int32)])
        for k in range(4):
            obuf[n, pl.ds(16 * k, 16)] = row

    pltpu.sync_copy(obuf, out_hbm.at[cid, pl.ds(sid * _RPT, _RPT)])


def _make_agg(nrows, d, psi_src=False):
    rpt = nrows // 16

    @functools.partial(
        pl.kernel,
        out_type=jax.ShapeDtypeStruct((2, nrows, d), _f32),
        mesh=_mesh,
        scratch_types=[
            pltpu.VMEM((CPT, 2, CHUNK), jnp.int32),
            [pltpu.VMEM((CHUNK, d), _f32)] * 8,
            [pltpu.SemaphoreType.DMA] * 8,
            [pltpu.SemaphoreType.DMA] * 8,
            pltpu.VMEM_SHARED((nrows, d), _f32),
        ],
        compiler_params=_sc_vec_params,
    )
    def _agg(y_hbm, ei_hbm, pad_hbm, zeros_hbm, out_hbm,
             eiv, bufs, gsems, ssems, acc):
        cid = lax.axis_index("c")
        sid = lax.axis_index("s")
        tid = cid * 16 + sid
        _stage_edges(ei_hbm, pad_hbm, eiv, tid)
        pltpu.sync_copy(zeros_hbm, acc.at[pl.ds(sid * rpt, rpt)])
        if psi_src:
            @pl.loop(0, CPT)
            def _(j):
                for l in range(8):
                    v = eiv[j, 0, pl.ds(16 * l, 16)]
                    eiv[j, 0, pl.ds(16 * l, 16)] = (
                        4 * (v >> 1) + (v & 1))
        plsc.subcore_barrier()

        for c in range(4):
            pltpu.async_copy(y_hbm.at[eiv.at[c, 0]], bufs[c], gsems[c])

        @pl.loop(0, CPT, step=8)
        def _(j):
            for k in range(8):
                b = k % 8
                pltpu.make_async_copy(y_hbm.at[eiv.at[j + k, 0]],
                                      bufs[b], gsems[b]).wait()
                pltpu.async_copy(bufs[b], acc.at[eiv.at[j + k, 1]],
                                 ssems[b], add=True)
                bn = (k + 4) % 8

                @pl.when(j + k + 4 < CPT)
                def _():
                    @pl.when(j + k >= 4)
                    def _():
                        pltpu.make_async_copy(
                            bufs[bn], acc.at[eiv.at[j + k - 4, 1]],
                            ssems[bn]).wait()

                    pltpu.async_copy(y_hbm.at[eiv.at[j + k + 4, 0]],
                                     bufs[bn], gsems[bn])

        for c in range(CPT - 8, CPT):
            b = c % 8
            pltpu.make_async_copy(bufs[b], acc.at[eiv.at[c, 1]],
                                  ssems[b]).wait()

        plsc.subcore_barrier()
        pltpu.sync_copy(
            acc.at[pl.ds(sid * rpt, rpt)],
            out_hbm.at[cid, pl.ds(sid * rpt, rpt)],
        )

    return _agg


_agg64 = _make_agg(NP, 64)
_agg32 = _make_agg(NP, 32, psi_src=True)


_R = 1024
_GRID = (NP // 2) // _R


def _disf(dpf_ref):
    degf = 1.0 + dpf_ref[0] + dpf_ref[1]
    return 1.0 / jnp.sqrt(degf)


def _dense1_body(xf_ref, wbd_ref, dpf_ref, y_ref, s_ref):
    xwf = jnp.dot(xf_ref[...], wbd_ref[...], preferred_element_type=_f32)
    disf = _disf(dpf_ref)
    y_ref[...] = xwf * disf
    s_ref[...] = xwf * (disf * disf)


def _dense1(xf, w1bd, dpf):
    return pl.pallas_call(
        _dense1_body,
        grid=(_GRID,),
        in_specs=[
            pl.BlockSpec((_R, 512), lambda i: (i, 0)),
            pl.BlockSpec((512, 128), lambda i: (0, 0)),
            pl.BlockSpec((2, _R, 128), lambda i: (0, i, 0)),
        ],
        out_specs=[
            pl.BlockSpec((_R, 128), lambda i: (i, 0)),
            pl.BlockSpec((_R, 128), lambda i: (i, 0)),
        ],
        out_shape=[
            jax.ShapeDtypeStruct((NP // 2, 128), _f32),
            jax.ShapeDtypeStruct((NP // 2, 128), _f32),
        ],
    )(xf, w1bd, dpf)


def _dense2_body(af_ref, sf_ref, dpf_ref, wbd_ref, bf_ref, yz_ref):
    disf = _disf(dpf_ref)
    aggf = af_ref[0] + af_ref[1]
    hf = jnp.maximum(aggf * disf + sf_ref[...] + bf_ref[...], 0.0)
    xw2 = jnp.dot(hf, wbd_ref[...], preferred_element_type=_f32)
    dis32 = jnp.concatenate([disf[:, 0:32], disf[:, 64:96]], axis=1)
    y2 = xw2 * dis32
    s2 = xw2 * (dis32 * dis32)
    yz_ref[...] = jnp.concatenate([y2, s2], axis=1)


def _dense2(af, sf, dpf, w2bd, b1f):
    return pl.pallas_call(
        _dense2_body,
        grid=(_GRID,),
        in_specs=[
            pl.BlockSpec((2, _R, 128), lambda i: (0, i, 0)),
            pl.BlockSpec((_R, 128), lambda i: (i, 0)),
            pl.BlockSpec((2, _R, 128), lambda i: (0, i, 0)),
            pl.BlockSpec((128, 64), lambda i: (0, 0)),
            pl.BlockSpec((1, 128), lambda i: (0, 0)),
        ],
        out_specs=pl.BlockSpec((_R, 128), lambda i: (i, 0)),
        out_shape=jax.ShapeDtypeStruct((NP // 2, 128), _f32),
    )(af, sf, dpf, w2bd, b1f)


def _dense3_body(af_ref, yz4_ref, dp4_ref, wbd_ref, bf_ref, bfc_ref, o_ref):
    deg4 = 1.0 + dp4_ref[0] + dp4_ref[1]
    dis4 = 1.0 / jnp.sqrt(deg4)
    dis32 = jnp.concatenate(
        [dis4[:, 0:32], dis4[:, 64:96], dis4[:, 128:160], dis4[:, 192:224]],
        axis=1)
    yz4 = yz4_ref[...]
    s2 = jnp.concatenate([yz4[:, 64:128], yz4[:, 192:256]], axis=1)
    a2 = af_ref[0] + af_ref[1]
    hf = jnp.maximum(a2 * dis32 + s2 + bf_ref[...], 0.0)
    o_ref[...] = (jnp.dot(hf, wbd_ref[...], preferred_element_type=_f32)
                  + bfc_ref[...])


def _dense3(af, yz4, dp4, wfcbd, b2f, bfc):
    return pl.pallas_call(
        _dense3_body,
        grid=(_GRID,),
        in_specs=[
            pl.BlockSpec((2, _R // 2, 128), lambda i: (0, i, 0)),
            pl.BlockSpec((_R // 2, 256), lambda i: (i, 0)),
            pl.BlockSpec((2, _R // 2, 256), lambda i: (0, i, 0)),
            pl.BlockSpec((128, 4), lambda i: (0, 0)),
            pl.BlockSpec((1, 128), lambda i: (0, 0)),
            pl.BlockSpec((1, 1), lambda i: (0, 0)),
        ],
        out_specs=pl.BlockSpec((_R // 2, 4), lambda i: (i, 0)),
        out_shape=jax.ShapeDtypeStruct((NP // 4, 4), _f32),
    )(af, yz4, dp4, wfcbd, b2f, bfc)


def kernel(x, edge_index, W1, b1, W2, b2, Wfc, bfc):
    ei = jnp.swapaxes(edge_index.astype(jnp.int32).reshape(2, ECH, CHUNK),
                      0, 1)
    padc = jnp.asarray(_PAD_NP)

    znp = jnp.zeros((NP,), _f32)
    z64 = jnp.zeros((NP // 16, 64), _f32)
    z32 = jnp.zeros((NP // 16, 32), _f32)

    w1bd = jnp.zeros((512, 128), _f32).at[:256, :64].set(W1).at[256:, 64:].set(W1)
    w2bd = jnp.zeros((128, 64), _f32).at[:64, :32].set(W2).at[64:, 32:].set(W2)
    wfcbd = jnp.zeros((128, 4), _f32)
    for k in range(4):
        wfcbd = wfcbd.at[32 * k:32 * (k + 1), k].set(Wfc[:, 0])
    b1f = jnp.tile(b1, 2).reshape(1, 128)
    b2f = jnp.tile(b2, 4).reshape(1, 128)

    xf = x.reshape(N // 2, 512)

    dp = _deg_sc(ei, padc, znp)
    dpf = dp.reshape(2, NP // 2, 128)
    y1f, s1f = _dense1(xf, w1bd, dpf)
    a1 = _agg64(y1f.reshape(NP, 64), ei, padc, z64)
    yz = _dense2(a1.reshape(2, NP // 2, 128), s1f, dpf, w2bd, b1f)
    a2 = _agg32(yz.reshape(2 * NP, 32), ei, padc, z32)
    o = _dense3(a2.reshape(2, NP // 4, 128), yz.reshape(NP // 4, 256),
                dp.reshape(2, NP // 4, 256), wfcbd, b2f,
                bfc.reshape(1, 1))
    return o.reshape(NP, 1)[:N]
